# baseline (speedup 1.0000x reference)
import functools

import jax
import jax.numpy as jnp
from jax import lax
from jax.experimental import pallas as pl
from jax.experimental.pallas import tpu as pltpu
from jax.experimental.pallas import tpu_sc as plsc

NROWS = 100000
BATCH = 16384
D = 32
NC = 2
NS = 16
NW = NC * NS
BPW = BATCH // NW
CH = 128
SUB = 2
NRD = BPW // (SUB * CH)

RB = 8192
RGRID = (NROWS + RB - 1) // RB


def _repack_body(mu_r, mi_r, pu_r, pi_r, w2a_r, o_r):
    dg = functools.partial(lax.dot_general,
                           dimension_numbers=(((0,), (0,)), ((), ())),
                           preferred_element_type=jnp.float32)
    eye = jnp.eye(D, dtype=jnp.bfloat16)
    c0 = (mu_r[:] * w2a_r[:]).astype(jnp.bfloat16).T
    c1 = mi_r[:].astype(jnp.bfloat16).T
    c2 = dg(pu_r[:].astype(jnp.bfloat16), eye).astype(jnp.bfloat16)
    c3 = dg(pi_r[:].astype(jnp.bfloat16), eye).astype(jnp.bfloat16)
    o_r[:] = pltpu.bitcast(jnp.concatenate([c0, c1, c2, c3], axis=1),
                           jnp.int32)


def _repack(mu_t, mi_t, pu_t, pi_t, w2a):
    col_spec = pl.BlockSpec((D, RB), lambda b: (0, b))

    def whole(a):
        return pl.BlockSpec(a.shape, lambda b: tuple(0 for _ in a.shape))

    return pl.pallas_call(
        _repack_body,
        grid=(RGRID,),
        in_specs=[col_spec, col_spec, col_spec, col_spec, whole(w2a)],
        out_specs=pl.BlockSpec((RB // 2, 4 * D), lambda b: (b, 0)),
        out_shape=jax.ShapeDtypeStruct((NROWS // 2, 4 * D), jnp.int32),
    )(mu_t, mi_t, pu_t, pi_t, w2a)


def _sc_gather(u3, i3, table):
    mesh = plsc.VectorSubcoreMesh(core_axis_name="c", subcore_axis_name="s")
    out_t = [jax.ShapeDtypeStruct((BATCH, 128), jnp.int32)
             for _ in range(2)]
    scratch = [
        pltpu.VMEM((NRD, SUB, CH), jnp.int32),
        pltpu.VMEM((NRD, SUB, CH), jnp.int32),
        pltpu.VMEM((SUB, CH, 128), jnp.int32),
        pltpu.VMEM((SUB, CH, 128), jnp.int32),
        pltpu.SemaphoreType.DMA,
        pltpu.SemaphoreType.DMA,
    ]

    @functools.partial(pl.kernel, out_type=out_t, mesh=mesh,
                       scratch_types=scratch)
    def k(u_h, i_h, t_h, ou, oi, uv, iv, gu, gi, sem, wsem):
        wid = lax.axis_index("s") * NC + lax.axis_index("c")
        base = wid * BPW
        pltpu.sync_copy(u_h.at[wid], uv)
        pltpu.sync_copy(i_h.at[wid], iv)
        for r in range(NRD):
            cs = []
            for s in range(SUB):
                cs.append(pltpu.async_copy(
                    t_h.at[uv.at[r, s]], gu.at[s], sem))
                cs.append(pltpu.async_copy(
                    t_h.at[iv.at[r, s]], gi.at[s], sem))
            for c in cs:
                c.wait()
            ws = []
            for s in range(SUB):
                ro = base + (r * SUB + s) * CH
                ws.append(pltpu.async_copy(
                    gu.at[s], ou.at[pl.ds(ro, CH)], wsem))
                ws.append(pltpu.async_copy(
                    gi.at[s], oi.at[pl.ds(ro, CH)], wsem))
            for w in ws:
                w.wait()

    return k(u3, i3, table)


BLK = 4096


def _unpack(g32, par):
    gu32 = lax.bitcast_convert_type(g32, jnp.uint32)
    lo = lax.bitcast_convert_type(
        (gu32 & jnp.uint32(0xFFFF)).astype(jnp.uint16), jnp.bfloat16)
    hi = lax.bitcast_convert_type(
        (gu32 >> jnp.uint32(16)).astype(jnp.uint16), jnp.bfloat16)
    return jnp.where(par != 0, hi, lo).astype(jnp.float32)


def _tc_body(gu_r, gi_r, inp_r, W0_r, b0_r, W1_r, b1_r, Wl_r, bl_r,
             w2b_r, b2_r, o_r):
    dot = functools.partial(jnp.dot, preferred_element_type=jnp.float32)
    pu = inp_r[:, 0:1] & 1
    pi = inp_r[:, 1:2] & 1
    mfu_s = _unpack(gu_r[:, 0:D], pu)
    mfi = _unpack(gi_r[:, D:2 * D], pi)
    mlpu = _unpack(gu_r[:, 2 * D:3 * D], pu)
    mlpi = _unpack(gi_r[:, 3 * D:4 * D], pi)
    x = dot(mlpu, W0_r[0:D, :]) + dot(mlpi, W0_r[D:2 * D, :])
    x = jnp.maximum(x + b0_r[:], 0.0)
    x = jnp.maximum(dot(x, W1_r[:]) + b1_r[:], 0.0)
    mlp_vec = dot(x, Wl_r[:]) + bl_r[:]
    logits = (jnp.sum(mfu_s * mfi, axis=1)
              + jnp.sum(mlp_vec * w2b_r[:], axis=1) + b2_r[0, 0])
    o_r[:] = jax.nn.sigmoid(logits)


def _tc_mlp(gu, gi, inputs, W0, b0, W1, b1, Wl, bl, w2b, b2):
    grid = (BATCH // BLK,)
    row_spec = pl.BlockSpec((BLK, 128), lambda b: (b, 0))
    inp_spec = pl.BlockSpec((BLK, 2), lambda b: (b, 0))

    def whole(a):
        return pl.BlockSpec(a.shape, lambda b: tuple(0 for _ in a.shape))

    return pl.pallas_call(
        _tc_body,
        grid=grid,
        in_specs=[row_spec, row_spec, inp_spec,
                  whole(W0), whole(b0), whole(W1), whole(b1),
                  whole(Wl), whole(bl), whole(w2b), whole(b2)],
        out_specs=pl.BlockSpec((BLK,), lambda b: (b,)),
        out_shape=jax.ShapeDtypeStruct((BATCH,), jnp.float32),
    )(gu, gi, inputs, W0, b0, W1, b1, Wl, bl, w2b, b2)


def kernel(inputs, mf_user, mf_item, mlp_user, mlp_item,
           W0, b0, W1, b1, Wl, bl, W2, b2):
    w2a = W2[0:D, 0].reshape(D, 1)
    w2b = W2[D:2 * D, 0].reshape(1, D)
    t32 = _repack(mf_user.T, mf_item.T, mlp_user.T, mlp_item.T, w2a)
    u = inputs[:, 0]
    i = inputs[:, 1]
    u3 = (u >> 1).reshape(NW, NRD, SUB, CH)
    i3 = (i >> 1).reshape(NW, NRD, SUB, CH)
    gu, gi = _sc_gather(u3, i3, t32)
    return _tc_mlp(gu, gi, inputs,
                   W0, b0.reshape(1, -1), W1, b1.reshape(1, -1),
                   Wl, bl.reshape(1, -1), w2b, b2.reshape(1, 1))


# speedup vs baseline: 1.0152x; 1.0152x over previous
import functools

import jax
import jax.numpy as jnp
from jax import lax
from jax.experimental import pallas as pl
from jax.experimental.pallas import tpu as pltpu
from jax.experimental.pallas import tpu_sc as plsc

NROWS = 100000
BATCH = 16384
HALF = BATCH // 2
D = 32
NC = 2
NS = 16
NW = NC * NS
BPW = HALF // NW
CH = 128
SUB = 2
NRD = BPW // (SUB * CH)

RB = 8192
RGRID = (NROWS + RB - 1) // RB


def _repack_body(mu_r, mi_r, pu_r, pi_r, w2a_r, o_r):
    c0 = (mu_r[:] * w2a_r[:]).astype(jnp.bfloat16).T
    c1 = mi_r[:].astype(jnp.bfloat16).T
    c2 = pu_r[:].astype(jnp.bfloat16).T
    c3 = pi_r[:].astype(jnp.bfloat16).T
    o_r[:] = pltpu.bitcast(jnp.concatenate([c0, c1, c2, c3], axis=1),
                           jnp.int32)


def _repack(mu_t, mi_t, pu_t, pi_t, w2a):
    col_spec = pl.BlockSpec((D, RB), lambda b: (0, b))

    def whole(a):
        return pl.BlockSpec(a.shape, lambda b: tuple(0 for _ in a.shape))

    return pl.pallas_call(
        _repack_body,
        grid=(RGRID,),
        in_specs=[col_spec, col_spec, col_spec, col_spec, whole(w2a)],
        out_specs=pl.BlockSpec((RB // 2, 4 * D), lambda b: (b, 0)),
        out_shape=jax.ShapeDtypeStruct((NROWS // 2, 4 * D), jnp.int32),
    )(mu_t, mi_t, pu_t, pi_t, w2a)


def _sc_gather(u3, i3, table):
    mesh = plsc.VectorSubcoreMesh(core_axis_name="c", subcore_axis_name="s")
    out_t = [jax.ShapeDtypeStruct((HALF, 128), jnp.int32)
             for _ in range(2)]
    scratch = [
        pltpu.VMEM((NRD, SUB, CH), jnp.int32),
        pltpu.VMEM((NRD, SUB, CH), jnp.int32),
        pltpu.VMEM((SUB, CH, 128), jnp.int32),
        pltpu.VMEM((SUB, CH, 128), jnp.int32),
        pltpu.SemaphoreType.DMA,
        pltpu.SemaphoreType.DMA,
    ]

    @functools.partial(pl.kernel, out_type=out_t, mesh=mesh,
                       scratch_types=scratch)
    def k(u_h, i_h, t_h, ou, oi, uv, iv, gu, gi, sem, wsem):
        wid = lax.axis_index("s") * NC + lax.axis_index("c")
        base = wid * BPW
        pltpu.sync_copy(u_h.at[wid], uv)
        pltpu.sync_copy(i_h.at[wid], iv)
        for r in range(NRD):
            cs = []
            for s in range(SUB):
                cs.append(pltpu.async_copy(
                    t_h.at[uv.at[r, s]], gu.at[s], sem))
                cs.append(pltpu.async_copy(
                    t_h.at[iv.at[r, s]], gi.at[s], sem))
            for c in cs:
                c.wait()
            ws = []
            for s in range(SUB):
                ro = base + (r * SUB + s) * CH
                ws.append(pltpu.async_copy(
                    gu.at[s], ou.at[pl.ds(ro, CH)], wsem))
                ws.append(pltpu.async_copy(
                    gi.at[s], oi.at[pl.ds(ro, CH)], wsem))
            for w in ws:
                w.wait()

    return k(u3, i3, table)


BLK = 4096


def _unpack(g32, par):
    gu32 = lax.bitcast_convert_type(g32, jnp.uint32)
    lo = lax.bitcast_convert_type(
        (gu32 & jnp.uint32(0xFFFF)).astype(jnp.uint16), jnp.bfloat16)
    hi = lax.bitcast_convert_type(
        (gu32 >> jnp.uint32(16)).astype(jnp.uint16), jnp.bfloat16)
    return jnp.where(par != 0, hi, lo).astype(jnp.float32)


def _tc_body(gu_r, gi_r, inp_r, W0_r, b0_r, W1_r, b1_r, Wl_r, bl_r,
             w2b_r, b2_r, o_r):
    dot = functools.partial(jnp.dot, preferred_element_type=jnp.float32)
    pu = inp_r[:, 0:1] & 1
    pi = inp_r[:, 1:2] & 1
    mfu_s = _unpack(gu_r[:, 0:D], pu)
    mfi = _unpack(gi_r[:, D:2 * D], pi)
    mlpu = _unpack(gu_r[:, 2 * D:3 * D], pu)
    mlpi = _unpack(gi_r[:, 3 * D:4 * D], pi)
    x = dot(mlpu, W0_r[0:D, :]) + dot(mlpi, W0_r[D:2 * D, :])
    x = jnp.maximum(x + b0_r[:], 0.0)
    x = jnp.maximum(dot(x, W1_r[:]) + b1_r[:], 0.0)
    mlp_vec = dot(x, Wl_r[:]) + bl_r[:]
    logits = (jnp.sum(mfu_s * mfi, axis=1)
              + jnp.sum(mlp_vec * w2b_r[:], axis=1) + b2_r[0, 0])
    o_r[:] = jax.nn.sigmoid(logits)


def _tc_mlp(gu, gi, inputs, W0, b0, W1, b1, Wl, bl, w2b, b2):
    grid = (HALF // BLK,)
    row_spec = pl.BlockSpec((BLK, 128), lambda b: (b, 0))
    inp_spec = pl.BlockSpec((BLK, 2), lambda b: (b, 0))

    def whole(a):
        return pl.BlockSpec(a.shape, lambda b: tuple(0 for _ in a.shape))

    return pl.pallas_call(
        _tc_body,
        grid=grid,
        in_specs=[row_spec, row_spec, inp_spec,
                  whole(W0), whole(b0), whole(W1), whole(b1),
                  whole(Wl), whole(bl), whole(w2b), whole(b2)],
        out_specs=pl.BlockSpec((BLK,), lambda b: (b,)),
        out_shape=jax.ShapeDtypeStruct((HALF,), jnp.float32),
    )(gu, gi, inputs, W0, b0, W1, b1, Wl, bl, w2b, b2)


def kernel(inputs, mf_user, mf_item, mlp_user, mlp_item,
           W0, b0, W1, b1, Wl, bl, W2, b2):
    w2a = W2[0:D, 0].reshape(D, 1)
    w2b = W2[D:2 * D, 0].reshape(1, D)
    t32 = _repack(mf_user.T, mf_item.T, mlp_user.T, mlp_item.T, w2a)
    u = inputs[:, 0]
    i = inputs[:, 1]
    uq = (u >> 1).reshape(2, NW, NRD, SUB, CH)
    iq = (i >> 1).reshape(2, NW, NRD, SUB, CH)
    wargs = (W0, b0.reshape(1, -1), W1, b1.reshape(1, -1),
             Wl, bl.reshape(1, -1), w2b, b2.reshape(1, 1))
    outs = []
    for h in range(2):
        gu, gi = _sc_gather(uq[h], iq[h], t32)
        outs.append(_tc_mlp(gu, gi, inputs[h * HALF:(h + 1) * HALF],
                            *wargs))
    return jnp.concatenate(outs)


# speedup vs baseline: 1.0351x; 1.0196x over previous
import functools

import jax
import jax.numpy as jnp
from jax import lax
from jax.experimental import pallas as pl
from jax.experimental.pallas import tpu as pltpu
from jax.experimental.pallas import tpu_sc as plsc

NROWS = 100000
BATCH = 16384
HALF = BATCH // 2
D = 32
NC = 2
NS = 16
NW = NC * NS
BPW = HALF // NW
CH = 128
SUB = 2
NRD = BPW // (SUB * CH)

RB = 8192
RGRID = (NROWS + RB - 1) // RB


def _repack_body(mu_r, mi_r, pu_r, pi_r, w2a_r, o_r):
    c0 = (mu_r[:] * w2a_r[:]).astype(jnp.bfloat16).T
    c1 = mi_r[:].astype(jnp.bfloat16).T
    c2 = pu_r[:].astype(jnp.bfloat16).T
    c3 = pi_r[:].astype(jnp.bfloat16).T
    o_r[:] = pltpu.bitcast(jnp.concatenate([c0, c1, c2, c3], axis=1),
                           jnp.int32)


def _repack(mu_t, mi_t, pu_t, pi_t, w2a):
    col_spec = pl.BlockSpec((D, RB), lambda b: (0, b))

    def whole(a):
        return pl.BlockSpec(a.shape, lambda b: tuple(0 for _ in a.shape))

    return pl.pallas_call(
        _repack_body,
        grid=(RGRID,),
        in_specs=[col_spec, col_spec, col_spec, col_spec, whole(w2a)],
        out_specs=pl.BlockSpec((RB // 2, 4 * D), lambda b: (b, 0)),
        out_shape=jax.ShapeDtypeStruct((NROWS // 2, 4 * D), jnp.int32),
    )(mu_t, mi_t, pu_t, pi_t, w2a)


def _sc_gather(u3, i3, table):
    mesh = plsc.VectorSubcoreMesh(core_axis_name="c", subcore_axis_name="s")
    out_t = [jax.ShapeDtypeStruct((HALF, 128), jnp.int32)
             for _ in range(2)]
    scratch = [
        pltpu.VMEM((NRD, SUB, CH), jnp.int32),
        pltpu.VMEM((NRD, SUB, CH), jnp.int32),
        pltpu.VMEM((SUB, CH, 128), jnp.int32),
        pltpu.VMEM((SUB, CH, 128), jnp.int32),
        pltpu.SemaphoreType.DMA,
        pltpu.SemaphoreType.DMA,
    ]

    @functools.partial(pl.kernel, out_type=out_t, mesh=mesh,
                       scratch_types=scratch)
    def k(u_h, i_h, t_h, ou, oi, uv, iv, gu, gi, sem, wsem):
        wid = lax.axis_index("s") * NC + lax.axis_index("c")
        base = wid * BPW
        pltpu.sync_copy(u_h.at[wid], uv)
        pltpu.sync_copy(i_h.at[wid], iv)
        for r in range(NRD):
            cs = []
            for s in range(SUB):
                cs.append(pltpu.async_copy(
                    t_h.at[uv.at[r, s]], gu.at[s], sem))
                cs.append(pltpu.async_copy(
                    t_h.at[iv.at[r, s]], gi.at[s], sem))
            for c in cs:
                c.wait()
            ws = []
            for s in range(SUB):
                ro = base + (r * SUB + s) * CH
                ws.append(pltpu.async_copy(
                    gu.at[s], ou.at[pl.ds(ro, CH)], wsem))
                ws.append(pltpu.async_copy(
                    gi.at[s], oi.at[pl.ds(ro, CH)], wsem))
            for w in ws:
                w.wait()

    return k(u3, i3, table)


BLK = 4096


def _unpack(g32, sh):
    gu32 = lax.bitcast_convert_type(g32, jnp.uint32)
    bits = ((gu32 >> sh) & jnp.uint32(0xFFFF)).astype(jnp.uint16)
    return lax.bitcast_convert_type(bits, jnp.bfloat16).astype(jnp.float32)


def _tc_body(gu_r, gi_r, inp_r, W0_r, b0_r, W1_r, b1_r, Wl_r, bl_r,
             w2b_r, b2_r, o_r):
    dot = functools.partial(jnp.dot, preferred_element_type=jnp.float32)
    shu = ((inp_r[:, 0:1] & 1) << 4).astype(jnp.uint32)
    shi = ((inp_r[:, 1:2] & 1) << 4).astype(jnp.uint32)
    mfu_s = _unpack(gu_r[:, 0:D], shu)
    mfi = _unpack(gi_r[:, D:2 * D], shi)
    mlpu = _unpack(gu_r[:, 2 * D:3 * D], shu)
    mlpi = _unpack(gi_r[:, 3 * D:4 * D], shi)
    x = dot(mlpu, W0_r[0:D, :]) + dot(mlpi, W0_r[D:2 * D, :])
    x = jnp.maximum(x + b0_r[:], 0.0)
    x = jnp.maximum(dot(x, W1_r[:]) + b1_r[:], 0.0)
    mlp_vec = dot(x, Wl_r[:]) + bl_r[:]
    logits = (jnp.sum(mfu_s * mfi, axis=1)
              + jnp.sum(mlp_vec * w2b_r[:], axis=1) + b2_r[0, 0])
    o_r[:] = jax.nn.sigmoid(logits)


def _tc_mlp(gu, gi, inputs, W0, b0, W1, b1, Wl, bl, w2b, b2):
    grid = (HALF // BLK,)
    row_spec = pl.BlockSpec((BLK, 128), lambda b: (b, 0))
    inp_spec = pl.BlockSpec((BLK, 2), lambda b: (b, 0))

    def whole(a):
        return pl.BlockSpec(a.shape, lambda b: tuple(0 for _ in a.shape))

    return pl.pallas_call(
        _tc_body,
        grid=grid,
        in_specs=[row_spec, row_spec, inp_spec,
                  whole(W0), whole(b0), whole(W1), whole(b1),
                  whole(Wl), whole(bl), whole(w2b), whole(b2)],
        out_specs=pl.BlockSpec((BLK,), lambda b: (b,)),
        out_shape=jax.ShapeDtypeStruct((HALF,), jnp.float32),
    )(gu, gi, inputs, W0, b0, W1, b1, Wl, bl, w2b, b2)


def kernel(inputs, mf_user, mf_item, mlp_user, mlp_item,
           W0, b0, W1, b1, Wl, bl, W2, b2):
    w2a = W2[0:D, 0].reshape(D, 1)
    w2b = W2[D:2 * D, 0].reshape(1, D)
    t32 = _repack(mf_user.T, mf_item.T, mlp_user.T, mlp_item.T, w2a)
    u = inputs[:, 0]
    i = inputs[:, 1]
    uq = (u >> 1).reshape(2, NW, NRD, SUB, CH)
    iq = (i >> 1).reshape(2, NW, NRD, SUB, CH)
    wargs = (W0, b0.reshape(1, -1), W1, b1.reshape(1, -1),
             Wl, bl.reshape(1, -1), w2b, b2.reshape(1, 1))
    outs = []
    for h in range(2):
        gu, gi = _sc_gather(uq[h], iq[h], t32)
        outs.append(_tc_mlp(gu, gi, inputs[h * HALF:(h + 1) * HALF],
                            *wargs))
    return jnp.concatenate(outs)


# speedup vs baseline: 1.0387x; 1.0035x over previous
import functools

import jax
import jax.numpy as jnp
from jax import lax
from jax.experimental import pallas as pl
from jax.experimental.pallas import tpu as pltpu
from jax.experimental.pallas import tpu_sc as plsc

NROWS = 100000
BATCH = 16384
HALF = BATCH // 2
D = 32
NC = 2
NS = 16
NW = NC * NS
BPW = HALF // NW
CH = 128
SUB = 2
NRD = BPW // (SUB * CH)

RB = 8192
RGRID = (NROWS + RB - 1) // RB


def _repack_body(mu_r, mi_r, pu_r, pi_r, w2a_r, o_r):
    c0 = (mu_r[:] * w2a_r[:]).astype(jnp.bfloat16).T
    c1 = mi_r[:].astype(jnp.bfloat16).T
    c2 = pu_r[:].astype(jnp.bfloat16).T
    c3 = pi_r[:].astype(jnp.bfloat16).T
    o_r[:] = pltpu.bitcast(jnp.concatenate([c0, c1, c2, c3], axis=1),
                           jnp.int32)


def _repack(mu_t, mi_t, pu_t, pi_t, w2a):
    col_spec = pl.BlockSpec((D, RB), lambda b: (0, b))

    def whole(a):
        return pl.BlockSpec(a.shape, lambda b: tuple(0 for _ in a.shape))

    return pl.pallas_call(
        _repack_body,
        grid=(RGRID,),
        in_specs=[col_spec, col_spec, col_spec, col_spec, whole(w2a)],
        out_specs=pl.BlockSpec((RB // 2, 4 * D), lambda b: (b, 0)),
        out_shape=jax.ShapeDtypeStruct((NROWS // 2, 4 * D), jnp.int32),
    )(mu_t, mi_t, pu_t, pi_t, w2a)


def _sc_gather(u3, i3, table):
    mesh = plsc.VectorSubcoreMesh(core_axis_name="c", subcore_axis_name="s")
    out_t = [jax.ShapeDtypeStruct((HALF, 128), jnp.int32)
             for _ in range(2)]
    scratch = [
        pltpu.VMEM((NRD, SUB, CH), jnp.int32),
        pltpu.VMEM((NRD, SUB, CH), jnp.int32),
        pltpu.VMEM((SUB, CH, 128), jnp.int32),
        pltpu.VMEM((SUB, CH, 128), jnp.int32),
        pltpu.SemaphoreType.DMA,
        pltpu.SemaphoreType.DMA,
    ]

    @functools.partial(pl.kernel, out_type=out_t, mesh=mesh,
                       scratch_types=scratch)
    def k(u_h, i_h, t_h, ou, oi, uv, iv, gu, gi, sem, wsem):
        wid = lax.axis_index("s") * NC + lax.axis_index("c")
        base = wid * BPW
        pltpu.sync_copy(u_h.at[wid], uv)
        pltpu.sync_copy(i_h.at[wid], iv)
        for r in range(NRD):
            cs = []
            for s in range(SUB):
                cs.append(pltpu.async_copy(
                    t_h.at[uv.at[r, s]], gu.at[s], sem))
                cs.append(pltpu.async_copy(
                    t_h.at[iv.at[r, s]], gi.at[s], sem))
            for c in cs:
                c.wait()
            ws = []
            for s in range(SUB):
                ro = base + (r * SUB + s) * CH
                ws.append(pltpu.async_copy(
                    gu.at[s], ou.at[pl.ds(ro, CH)], wsem))
                ws.append(pltpu.async_copy(
                    gi.at[s], oi.at[pl.ds(ro, CH)], wsem))
            for w in ws:
                w.wait()

    return k(u3, i3, table)


BLK = 4096


def _unpack(g32, sh):
    gu32 = lax.bitcast_convert_type(g32, jnp.uint32)
    bits = ((gu32 >> sh) & jnp.uint32(0xFFFF)).astype(jnp.uint16)
    return lax.bitcast_convert_type(bits, jnp.bfloat16)


def _tc_body(gu_r, gi_r, inp_r, W0_r, b0_r, W1_r, b1_r, Wl_r, bl_r,
             w2b_r, b2_r, o_r):
    dot = functools.partial(jnp.dot, preferred_element_type=jnp.float32)
    shu = ((inp_r[:, 0:1] & 1) << 4).astype(jnp.uint32)
    shi = ((inp_r[:, 1:2] & 1) << 4).astype(jnp.uint32)
    mfu_s = _unpack(gu_r[:, 0:D], shu)
    mfi = _unpack(gi_r[:, D:2 * D], shi)
    mlpu = _unpack(gu_r[:, 2 * D:3 * D], shu)
    mlpi = _unpack(gi_r[:, 3 * D:4 * D], shi)
    x = dot(mlpu, W0_r[0:D, :]) + dot(mlpi, W0_r[D:2 * D, :])
    x = jnp.maximum(x + b0_r[:], 0.0)
    x = jnp.maximum(dot(x.astype(jnp.bfloat16), W1_r[:]) + b1_r[:], 0.0)
    mlp_vec = dot(x.astype(jnp.bfloat16), Wl_r[:]) + bl_r[:]
    mf = (mfu_s * mfi).astype(jnp.float32)
    logits = (jnp.sum(mf, axis=1)
              + jnp.sum(mlp_vec * w2b_r[:], axis=1) + b2_r[0, 0])
    o_r[:] = jax.nn.sigmoid(logits)


def _tc_mlp(gu, gi, inputs, W0, b0, W1, b1, Wl, bl, w2b, b2):
    grid = (HALF // BLK,)
    row_spec = pl.BlockSpec((BLK, 128), lambda b: (b, 0))
    inp_spec = pl.BlockSpec((BLK, 2), lambda b: (b, 0))

    def whole(a):
        return pl.BlockSpec(a.shape, lambda b: tuple(0 for _ in a.shape))

    return pl.pallas_call(
        _tc_body,
        grid=grid,
        in_specs=[row_spec, row_spec, inp_spec,
                  whole(W0), whole(b0), whole(W1), whole(b1),
                  whole(Wl), whole(bl), whole(w2b), whole(b2)],
        out_specs=pl.BlockSpec((BLK,), lambda b: (b,)),
        out_shape=jax.ShapeDtypeStruct((HALF,), jnp.float32),
    )(gu, gi, inputs, W0, b0, W1, b1, Wl, bl, w2b, b2)


def kernel(inputs, mf_user, mf_item, mlp_user, mlp_item,
           W0, b0, W1, b1, Wl, bl, W2, b2):
    w2a = W2[0:D, 0].reshape(D, 1)
    w2b = W2[D:2 * D, 0].reshape(1, D)
    t32 = _repack(mf_user.T, mf_item.T, mlp_user.T, mlp_item.T, w2a)
    u = inputs[:, 0]
    i = inputs[:, 1]
    uq = (u >> 1).reshape(2, NW, NRD, SUB, CH)
    iq = (i >> 1).reshape(2, NW, NRD, SUB, CH)
    wargs = (W0.astype(jnp.bfloat16), b0.reshape(1, -1),
             W1.astype(jnp.bfloat16), b1.reshape(1, -1),
             Wl.astype(jnp.bfloat16), bl.reshape(1, -1),
             w2b, b2.reshape(1, 1))
    outs = []
    for h in range(2):
        gu, gi = _sc_gather(uq[h], iq[h], t32)
        outs.append(_tc_mlp(gu, gi, inputs[h * HALF:(h + 1) * HALF],
                            *wargs))
    return jnp.concatenate(outs)
